# baseline (speedup 1.0000x reference)
import functools

import jax
import jax.numpy as jnp
from jax import lax
from jax.experimental import pallas as pl
from jax.experimental.pallas import tpu as pltpu
from jax.experimental.pallas import tpu_sc as plsc

N = 10000
FEAT = 128
HID = 64
LAT = 64
NUM_PRED = 5
NCLS = 16
E = 320000
BM = 400


def _mm_body(a_ref, b_ref, o_ref):
    o_ref[...] = jnp.dot(a_ref[...], b_ref[...],
                         preferred_element_type=jnp.float32)


def _mm(a, b):
    n, k = a.shape
    _, m = b.shape
    return pl.pallas_call(
        _mm_body,
        grid=(n // BM,),
        in_specs=[pl.BlockSpec((BM, k), lambda i: (i, 0)),
                  pl.BlockSpec((k, m), lambda i: (0, 0))],
        out_specs=pl.BlockSpec((BM, m), lambda i: (i, 0)),
        out_shape=jax.ShapeDtypeStruct((n, m), jnp.float32),
    )(a, b)


def _adjmm_body(adj_ref, y_ref, b_ref, o_ref):
    acc = jnp.dot(adj_ref[...], y_ref[...],
                  preferred_element_type=jnp.float32)
    o_ref[...] = jnp.maximum(acc + b_ref[...], 0.0)


def _adjmm_relu(adj, y, bias):
    n = adj.shape[0]
    m = y.shape[1]
    return pl.pallas_call(
        _adjmm_body,
        grid=(n // BM,),
        in_specs=[pl.BlockSpec((BM, n), lambda i: (i, 0)),
                  pl.BlockSpec((n, m), lambda i: (0, 0)),
                  pl.BlockSpec((1, m), lambda i: (0, 0))],
        out_specs=pl.BlockSpec((BM, m), lambda i: (i, 0)),
        out_shape=jax.ShapeDtypeStruct((n, m), jnp.float32),
    )(adj, y, bias)


def _tail_body(x_ref, nz_ref, wr_ref, br_ref, g1w_ref, g1b_ref, g2w_ref,
               g2b_ref, gfw_ref, gfb_ref, deg_ref, gen_ref):
    x = x_ref[...]
    deg_ref[...] = jnp.maximum(
        jnp.dot(x, wr_ref[...], preferred_element_type=jnp.float32)
        + br_ref[...], 0.0)
    g = x + nz_ref[...]
    g = jnp.maximum(
        jnp.dot(g, g1w_ref[...], preferred_element_type=jnp.float32)
        + g1b_ref[...], 0.0)
    g = jnp.maximum(
        jnp.dot(g, g2w_ref[...], preferred_element_type=jnp.float32)
        + g2b_ref[...], 0.0)
    gen_ref[...] = jnp.tanh(
        jnp.dot(g, gfw_ref[...], preferred_element_type=jnp.float32)
        + gfb_ref[...])


def _tail(x, noise, Wr, br, G1W, G1b, G2W, G2b, GfW, Gfb):
    outs = (jax.ShapeDtypeStruct((N, 1), jnp.float32),
            jax.ShapeDtypeStruct((N, NUM_PRED * FEAT), jnp.float32))
    return pl.pallas_call(
        _tail_body,
        grid=(N // BM,),
        in_specs=[pl.BlockSpec((BM, LAT), lambda i: (i, 0)),
                  pl.BlockSpec((BM, LAT), lambda i: (i, 0)),
                  pl.BlockSpec((LAT, 1), lambda i: (0, 0)),
                  pl.BlockSpec((1, 1), lambda i: (0, 0)),
                  pl.BlockSpec((LAT, 256), lambda i: (0, 0)),
                  pl.BlockSpec((1, 256), lambda i: (0, 0)),
                  pl.BlockSpec((256, 2048), lambda i: (0, 0)),
                  pl.BlockSpec((1, 2048), lambda i: (0, 0)),
                  pl.BlockSpec((2048, NUM_PRED * FEAT), lambda i: (0, 0)),
                  pl.BlockSpec((1, NUM_PRED * FEAT), lambda i: (0, 0))],
        out_specs=(pl.BlockSpec((BM, 1), lambda i: (i, 0)),
                   pl.BlockSpec((BM, NUM_PRED * FEAT), lambda i: (i, 0))),
        out_shape=outs,
    )(x, noise, Wr, br, G1W, G1b, G2W, G2b, GfW, Gfb)


def _asm_body(s_ref, yt_ref, yg_ref, dk_ref, d_ref, b_ref, ht_ref, hg_ref,
              *, width):
    dk = dk_ref[...]
    yt = yt_ref[...]
    s = s_ref[...] + yt
    for j in range(NUM_PRED):
        ygj = yg_ref[:, j * width:(j + 1) * width]
        v = (dk > float(j)).astype(jnp.float32)
        s = s + v * ygj
        hg_ref[:, j * width:(j + 1) * width] = jnp.maximum(
            (ygj + v * yt) / (1.0 + v) + b_ref[...], 0.0)
    ht_ref[...] = jnp.maximum(s / d_ref[...] + b_ref[...], 0.0)


def _assemble(S, ytop, ygen, dkf, d, bias, width):
    outs = (jax.ShapeDtypeStruct((N, width), jnp.float32),
            jax.ShapeDtypeStruct((N, NUM_PRED * width), jnp.float32))
    return pl.pallas_call(
        functools.partial(_asm_body, width=width),
        grid=(N // BM,),
        in_specs=[pl.BlockSpec((BM, width), lambda i: (i, 0)),
                  pl.BlockSpec((BM, width), lambda i: (i, 0)),
                  pl.BlockSpec((BM, NUM_PRED * width), lambda i: (i, 0)),
                  pl.BlockSpec((BM, 1), lambda i: (i, 0)),
                  pl.BlockSpec((BM, 1), lambda i: (i, 0)),
                  pl.BlockSpec((1, width), lambda i: (0, 0))],
        out_specs=(pl.BlockSpec((BM, width), lambda i: (i, 0)),
                   pl.BlockSpec((BM, NUM_PRED * width), lambda i: (i, 0))),
        out_shape=outs,
    )(S, ytop, ygen, dkf, d, bias)


NDUM = 64
NPAD = 10112
EP = 655360
ROWS128 = EP // 128
TILES = 32
TROWS = ROWS128 // TILES
CHUNKS = TROWS // 4


def _edge_slots(edges):
    r = edges[:, 0].astype(jnp.int32)
    c = edges[:, 1].astype(jnp.int32)
    kmin = jnp.minimum(r, c)
    kmax = jnp.maximum(r, c)
    dirb = (r > c).astype(jnp.int32)
    key2 = (kmin * N + kmax) * 2 + dirb
    sk = jnp.sort(key2)
    pair = sk >> 1
    dir1 = (sk & 1) == 1
    pos = jnp.arange(E, dtype=jnp.int32)
    F1 = jnp.concatenate([jnp.array([True]), pair[1:] != pair[:-1]])
    F2 = jnp.concatenate([jnp.array([True]), sk[1:] != sk[:-1]])
    BIG = jnp.int32(E)
    s1 = jax.lax.cummax(jnp.where(F1, pos, -1))
    a1 = jnp.where(F1, pos, BIG)
    suf1 = jnp.flip(jax.lax.cummin(jnp.flip(a1)))
    e1 = jnp.concatenate([suf1[1:], jnp.array([E], jnp.int32)])
    am = jnp.where(F2 & dir1, pos, BIG)
    msuf = jnp.flip(jax.lax.cummin(jnp.flip(am)))
    m = jnp.minimum(msuf, e1)
    maxc_head = jnp.maximum(m - pos, e1 - m)
    maxc = jnp.take(maxc_head, s1)
    act1 = (pos - s1) < maxc
    kminS = pair // N
    kmaxS = pair % N
    act2 = act1 & (kminS != kmaxS)
    dummy = N + (pos % NDUM)
    srcA = kmaxS
    dstA = jnp.where(act1, kminS, dummy)
    srcB = kminS
    dstB = jnp.where(act2, kmaxS, dummy)
    npad = EP - 2 * E
    padi = jnp.arange(npad, dtype=jnp.int32)
    src = jnp.concatenate([srcA, srcB, jnp.zeros((npad,), jnp.int32)])
    dst = jnp.concatenate([dstA, dstB, N + padi % NDUM])
    return src.reshape(ROWS128, 128), dst.reshape(ROWS128, 128)


def _make_spmm(width, with_d):
    mesh = plsc.VectorSubcoreMesh(core_axis_name="c", subcore_axis_name="s")
    out_type = [jax.ShapeDtypeStruct((2, NPAD, width), jnp.float32)]
    if with_d:
        out_type.append(jax.ShapeDtypeStruct((2 * NPAD,), jnp.float32))
    scratch = [pltpu.VMEM((2, 4, 128), jnp.int32),
               pltpu.VMEM((2, 4, 128), jnp.int32),
               pltpu.VMEM((2, 4, 128, width), jnp.float32),
               pltpu.VMEM((128,), jnp.float32),
               pltpu.VMEM((8, width), jnp.float32),
               pltpu.VMEM((640,), jnp.float32),
               pltpu.VMEM_SHARED((NPAD, width), jnp.float32),
               pltpu.VMEM_SHARED((NPAD,), jnp.float32),
               pltpu.SemaphoreType.DMA]
    vw = width // 16

    @functools.partial(
        pl.kernel, mesh=mesh, out_type=out_type, scratch_types=scratch,
        compiler_params=pltpu.CompilerParams(use_tc_tiling_on_sc=False))
    def spmm(src_h, dst_h, y_h, S_o, *rest):
        if with_d:
            d_o, srcv, dstv, rows, onesv, stage, dlin, Ssh, dsh, gsem = rest
        else:
            srcv, dstv, rows, onesv, stage, dlin, Ssh, dsh, gsem = rest
        cid = lax.axis_index("c")
        sid = lax.axis_index("s")
        rp = NPAD // 16
        s0 = sid * rp
        z16 = jnp.zeros((16,), jnp.float32)

        def zstage(i, carry):
            stage[i // vw, pl.ds((i % vw) * 16, 16)] = z16
            return carry
        lax.fori_loop(0, 8 * vw, zstage, 0)

        def zdlin(i, carry):
            dlin[pl.ds(i * 16, 16)] = z16
            return carry
        lax.fori_loop(0, 40, zdlin, 0)

        def zones(i, carry):
            onesv[pl.ds(i * 16, 16)] = jnp.ones((16,), jnp.float32)
            return carry
        lax.fori_loop(0, 8, zones, 0)

        def zS(k, carry):
            pltpu.sync_copy(stage, Ssh.at[pl.ds(s0 + k * 8, 8)])
            return carry
        lax.fori_loop(0, rp // 8, zS, 0)
        if with_d:
            pltpu.sync_copy(dlin.at[pl.ds(0, rp)], dsh.at[pl.ds(s0, rp)])
        plsc.subcore_barrier()
        base = (cid * 16 + sid) * TROWS

        def load(k, buf):
            r0 = base + k * 4
            pltpu.sync_copy(src_h.at[pl.ds(r0, 4)], srcv.at[buf])
            pltpu.sync_copy(dst_h.at[pl.ds(r0, 4)], dstv.at[buf])
            for j in range(4):
                pltpu.async_copy(y_h.at[srcv.at[buf, j]],
                                 rows.at[buf, j], gsem)

        load(0, 0)

        def chunk(k, carry):
            pb = lax.rem(k, 2)
            nb = lax.rem(k + 1, 2)

            @pl.when(k + 1 < CHUNKS)
            def _():
                load(k + 1, nb)

            for j in range(4):
                pltpu.make_async_copy(y_h.at[srcv.at[pb, j]],
                                      rows.at[pb, j], gsem).wait()
            for j in range(4):
                pltpu.sync_copy(rows.at[pb, j], Ssh.at[dstv.at[pb, j]],
                                add=True)
                if with_d:
                    pltpu.sync_copy(onesv, dsh.at[dstv.at[pb, j]], add=True)
            return carry

        lax.fori_loop(0, CHUNKS, chunk, 0)
        plsc.subcore_barrier()

        def outS(k, carry):
            pltpu.sync_copy(Ssh.at[pl.ds(s0 + k * 8, 8)], stage)
            pltpu.sync_copy(stage, S_o.at[cid, pl.ds(s0 + k * 8, 8)])
            return carry
        lax.fori_loop(0, rp // 8, outS, 0)
        if with_d:
            pltpu.sync_copy(dsh.at[pl.ds(s0, rp)], dlin.at[pl.ds(0, rp)])
            pltpu.sync_copy(dlin.at[pl.ds(0, rp)],
                            d_o.at[pl.ds(cid * NPAD + s0, rp)])

    return spmm


_SPMM_CACHE = {}


def _spmm(width, with_d):
    key = (width, with_d)
    if key not in _SPMM_CACHE:
        _SPMM_CACHE[key] = _make_spmm(width, with_d)
    return _SPMM_CACHE[key]


def kernel(feat, edges, adj, W1, b1, W2, b2, Wr, br, G1W, G1b, G2W, G2b,
           GfW, Gfb, C1W, C1b, C2W, C2b):
    row = lambda v: v.reshape(1, -1)
    y0 = _mm(feat, W1)
    x1 = _adjmm_relu(adj, y0, row(b1))
    z1 = _mm(x1, W2)
    x = _adjmm_relu(adj, z1, row(b2))
    noise = jax.random.normal(jax.random.key(7), (N, LAT), jnp.float32)
    degree, gen_feat = _tail(x, noise, Wr, row(br), G1W, row(G1b),
                             G2W, row(G2b), GfW, row(Gfb))
    deg_k = jnp.clip(degree.reshape(-1), 0, NUM_PRED).astype(jnp.int32)
    dkf = deg_k.astype(jnp.float32).reshape(N, 1)
    src, dst = _edge_slots(edges)
    fill = jnp.concatenate([feat, gen_feat.reshape(-1, FEAT)], axis=0)
    y1 = _mm(fill, C1W)
    y1top, y1gen = y1[:N], y1[N:].reshape(N, NUM_PRED * HID)
    S1p, d1p = _spmm(HID, True)(src, dst, y1top)
    S1 = (S1p[0] + S1p[1])[:N]
    d1p = d1p.reshape(2, NPAD)
    d = ((d1p[0] + d1p[1])[:N] + 1.0
         + deg_k.astype(jnp.float32)).reshape(N, 1)
    h1top, h1gen = _assemble(S1, y1top, y1gen, dkf, d, row(C1b), HID)
    h1 = jnp.concatenate([h1top, h1gen.reshape(-1, HID)], axis=0)
    y2 = _mm(h1, C2W)
    y2top, y2gen = y2[:N], y2[N:].reshape(N, NUM_PRED * NCLS)
    S2p, = _spmm(NCLS, False)(src, dst, y2top)
    S2 = (S2p[0] + S2p[1])[:N]
    h2top, h2gen = _assemble(S2, y2top, y2gen, dkf, d, row(C2b), NCLS)
    h = jnp.concatenate([h2top, h2gen.reshape(-1, NCLS)], axis=0)
    return (degree, gen_feat, h)


# speedup vs baseline: 1.0717x; 1.0717x over previous
import functools

import jax
import jax.numpy as jnp
from jax import lax
from jax.experimental import pallas as pl
from jax.experimental.pallas import tpu as pltpu
from jax.experimental.pallas import tpu_sc as plsc

N = 10000
FEAT = 128
HID = 64
LAT = 64
NUM_PRED = 5
NCLS = 16
E = 320000
BM = 400


def _mm_body(a_ref, b_ref, o_ref):
    o_ref[...] = jnp.dot(a_ref[...], b_ref[...],
                         preferred_element_type=jnp.float32)


def _mm(a, b):
    n, k = a.shape
    _, m = b.shape
    return pl.pallas_call(
        _mm_body,
        grid=(n // BM,),
        in_specs=[pl.BlockSpec((BM, k), lambda i: (i, 0)),
                  pl.BlockSpec((k, m), lambda i: (0, 0))],
        out_specs=pl.BlockSpec((BM, m), lambda i: (i, 0)),
        out_shape=jax.ShapeDtypeStruct((n, m), jnp.float32),
    )(a, b)


def _adjmm_body(adj_ref, y_ref, b_ref, o_ref):
    acc = jnp.dot(adj_ref[...], y_ref[...],
                  preferred_element_type=jnp.float32)
    o_ref[...] = jnp.maximum(acc + b_ref[...], 0.0)


def _adjmm_relu(adj, y, bias):
    n = adj.shape[0]
    m = y.shape[1]
    return pl.pallas_call(
        _adjmm_body,
        grid=(n // BM,),
        in_specs=[pl.BlockSpec((BM, n), lambda i: (i, 0)),
                  pl.BlockSpec((n, m), lambda i: (0, 0)),
                  pl.BlockSpec((1, m), lambda i: (0, 0))],
        out_specs=pl.BlockSpec((BM, m), lambda i: (i, 0)),
        out_shape=jax.ShapeDtypeStruct((n, m), jnp.float32),
    )(adj, y, bias)


def _tail_body(x_ref, nz_ref, wr_ref, br_ref, g1w_ref, g1b_ref, g2w_ref,
               g2b_ref, gfw_ref, gfb_ref, deg_ref, gen_ref):
    x = x_ref[...]
    deg_ref[...] = jnp.maximum(
        jnp.dot(x, wr_ref[...], preferred_element_type=jnp.float32)
        + br_ref[...], 0.0)
    g = x + nz_ref[...]
    g = jnp.maximum(
        jnp.dot(g, g1w_ref[...], preferred_element_type=jnp.float32)
        + g1b_ref[...], 0.0)
    g = jnp.maximum(
        jnp.dot(g, g2w_ref[...], preferred_element_type=jnp.float32)
        + g2b_ref[...], 0.0)
    gen_ref[...] = jnp.tanh(
        jnp.dot(g, gfw_ref[...], preferred_element_type=jnp.float32)
        + gfb_ref[...])


def _tail(x, noise, Wr, br, G1W, G1b, G2W, G2b, GfW, Gfb):
    outs = (jax.ShapeDtypeStruct((N, 1), jnp.float32),
            jax.ShapeDtypeStruct((N, NUM_PRED * FEAT), jnp.float32))
    return pl.pallas_call(
        _tail_body,
        grid=(N // BM,),
        in_specs=[pl.BlockSpec((BM, LAT), lambda i: (i, 0)),
                  pl.BlockSpec((BM, LAT), lambda i: (i, 0)),
                  pl.BlockSpec((LAT, 1), lambda i: (0, 0)),
                  pl.BlockSpec((1, 1), lambda i: (0, 0)),
                  pl.BlockSpec((LAT, 256), lambda i: (0, 0)),
                  pl.BlockSpec((1, 256), lambda i: (0, 0)),
                  pl.BlockSpec((256, 2048), lambda i: (0, 0)),
                  pl.BlockSpec((1, 2048), lambda i: (0, 0)),
                  pl.BlockSpec((2048, NUM_PRED * FEAT), lambda i: (0, 0)),
                  pl.BlockSpec((1, NUM_PRED * FEAT), lambda i: (0, 0))],
        out_specs=(pl.BlockSpec((BM, 1), lambda i: (i, 0)),
                   pl.BlockSpec((BM, NUM_PRED * FEAT), lambda i: (i, 0))),
        out_shape=outs,
    )(x, noise, Wr, br, G1W, G1b, G2W, G2b, GfW, Gfb)


def _asm_body(s_ref, yt_ref, yg_ref, dk_ref, d_ref, b_ref, ht_ref, hg_ref,
              *, width):
    dk = dk_ref[...]
    yt = yt_ref[...]
    s = s_ref[...] + yt
    for j in range(NUM_PRED):
        ygj = yg_ref[:, j * width:(j + 1) * width]
        v = (dk > float(j)).astype(jnp.float32)
        s = s + v * ygj
        hg_ref[:, j * width:(j + 1) * width] = jnp.maximum(
            (ygj + v * yt) / (1.0 + v) + b_ref[...], 0.0)
    ht_ref[...] = jnp.maximum(s / d_ref[...] + b_ref[...], 0.0)


def _assemble(S, ytop, ygen, dkf, d, bias, width):
    outs = (jax.ShapeDtypeStruct((N, width), jnp.float32),
            jax.ShapeDtypeStruct((N, NUM_PRED * width), jnp.float32))
    return pl.pallas_call(
        functools.partial(_asm_body, width=width),
        grid=(N // BM,),
        in_specs=[pl.BlockSpec((BM, width), lambda i: (i, 0)),
                  pl.BlockSpec((BM, width), lambda i: (i, 0)),
                  pl.BlockSpec((BM, NUM_PRED * width), lambda i: (i, 0)),
                  pl.BlockSpec((BM, 1), lambda i: (i, 0)),
                  pl.BlockSpec((BM, 1), lambda i: (i, 0)),
                  pl.BlockSpec((1, width), lambda i: (0, 0))],
        out_specs=(pl.BlockSpec((BM, width), lambda i: (i, 0)),
                   pl.BlockSpec((BM, NUM_PRED * width), lambda i: (i, 0))),
        out_shape=outs,
    )(S, ytop, ygen, dkf, d, bias)


NDUM = 64
NPAD = 10112
EP = 655360
ROWS128 = EP // 128
TILES = 32
TROWS = ROWS128 // TILES
CHUNKS = TROWS // 4


def _edge_slots(edges):
    r = edges[:, 0].astype(jnp.int32)
    c = edges[:, 1].astype(jnp.int32)
    kmin = jnp.minimum(r, c)
    kmax = jnp.maximum(r, c)
    dirb = (r > c).astype(jnp.int32)
    key2 = (kmin * N + kmax) * 2 + dirb
    sk = jnp.sort(key2)
    pair = sk >> 1
    dir1 = (sk & 1) == 1
    pos = jnp.arange(E, dtype=jnp.int32)
    F1 = jnp.concatenate([jnp.array([True]), pair[1:] != pair[:-1]])
    F2 = jnp.concatenate([jnp.array([True]), sk[1:] != sk[:-1]])
    BIG = jnp.int32(E)
    s1 = jax.lax.cummax(jnp.where(F1, pos, -1))
    a1 = jnp.where(F1, pos, BIG)
    suf1 = jnp.flip(jax.lax.cummin(jnp.flip(a1)))
    e1 = jnp.concatenate([suf1[1:], jnp.array([E], jnp.int32)])
    am = jnp.where(F2 & dir1, pos, BIG)
    msuf = jnp.flip(jax.lax.cummin(jnp.flip(am)))
    m = jnp.minimum(msuf, e1)
    maxc_head = jnp.maximum(m - pos, e1 - m)
    maxc = jnp.take(maxc_head, s1)
    act1 = (pos - s1) < maxc
    kminS = pair // N
    kmaxS = pair % N
    act2 = act1 & (kminS != kmaxS)
    dummy = N + (pos % NDUM)
    srcA = kmaxS
    dstA = jnp.where(act1, kminS, dummy)
    srcB = kminS
    dstB = jnp.where(act2, kmaxS, dummy)
    npad = EP - 2 * E
    padi = jnp.arange(npad, dtype=jnp.int32)
    src = jnp.concatenate([srcA, srcB, jnp.zeros((npad,), jnp.int32)])
    dst = jnp.concatenate([dstA, dstB, N + padi % NDUM])
    return src.reshape(ROWS128, 128), dst.reshape(ROWS128, 128)


def _make_spmm(width, with_d):
    mesh = plsc.VectorSubcoreMesh(core_axis_name="c", subcore_axis_name="s")
    out_type = [jax.ShapeDtypeStruct((2, NPAD, width), jnp.float32)]
    if with_d:
        out_type.append(jax.ShapeDtypeStruct((2 * NPAD,), jnp.float32))
    scratch = [pltpu.VMEM((2, 4, 128), jnp.int32),
               pltpu.VMEM((2, 4, 128), jnp.int32),
               pltpu.VMEM((2, 4, 128, width), jnp.float32),
               pltpu.VMEM((128,), jnp.float32),
               pltpu.VMEM((8, width), jnp.float32),
               pltpu.VMEM((640,), jnp.float32),
               pltpu.VMEM_SHARED((NPAD, width), jnp.float32),
               pltpu.VMEM_SHARED((NPAD,), jnp.float32),
               pltpu.SemaphoreType.DMA]
    vw = width // 16

    @functools.partial(
        pl.kernel, mesh=mesh, out_type=out_type, scratch_types=scratch,
        compiler_params=pltpu.CompilerParams(use_tc_tiling_on_sc=False))
    def spmm(src_h, dst_h, y_h, S_o, *rest):
        if with_d:
            d_o, srcv, dstv, rows, onesv, stage, dlin, Ssh, dsh, gsem = rest
        else:
            srcv, dstv, rows, onesv, stage, dlin, Ssh, dsh, gsem = rest
        cid = lax.axis_index("c")
        sid = lax.axis_index("s")
        rp = NPAD // 16
        s0 = sid * rp
        z16 = jnp.zeros((16,), jnp.float32)

        def zstage(i, carry):
            stage[i // vw, pl.ds((i % vw) * 16, 16)] = z16
            return carry
        lax.fori_loop(0, 8 * vw, zstage, 0)

        def zdlin(i, carry):
            dlin[pl.ds(i * 16, 16)] = z16
            return carry
        lax.fori_loop(0, 40, zdlin, 0)

        def zones(i, carry):
            onesv[pl.ds(i * 16, 16)] = jnp.ones((16,), jnp.float32)
            return carry
        lax.fori_loop(0, 8, zones, 0)

        def zS(k, carry):
            pltpu.sync_copy(stage, Ssh.at[pl.ds(s0 + k * 8, 8)])
            return carry
        lax.fori_loop(0, rp // 8, zS, 0)
        if with_d:
            pltpu.sync_copy(dlin.at[pl.ds(0, rp)], dsh.at[pl.ds(s0, rp)])
        plsc.subcore_barrier()
        base = (cid * 16 + sid) * TROWS

        def load(k, buf):
            r0 = base + k * 4
            pltpu.sync_copy(src_h.at[pl.ds(r0, 4)], srcv.at[buf])
            pltpu.sync_copy(dst_h.at[pl.ds(r0, 4)], dstv.at[buf])
            for j in range(4):
                pltpu.async_copy(y_h.at[srcv.at[buf, j]],
                                 rows.at[buf, j], gsem)

        load(0, 0)

        def chunk(k, carry):
            pb = lax.rem(k, 2)
            nb = lax.rem(k + 1, 2)

            @pl.when(k + 1 < CHUNKS)
            def _():
                load(k + 1, nb)

            for j in range(4):
                pltpu.make_async_copy(y_h.at[srcv.at[pb, j]],
                                      rows.at[pb, j], gsem).wait()
            for j in range(4):
                pltpu.sync_copy(rows.at[pb, j], Ssh.at[dstv.at[pb, j]],
                                add=True)
                if with_d:
                    pltpu.sync_copy(onesv, dsh.at[dstv.at[pb, j]], add=True)
            return carry

        lax.fori_loop(0, CHUNKS, chunk, 0)
        plsc.subcore_barrier()

        def outS(k, carry):
            pltpu.sync_copy(Ssh.at[pl.ds(s0 + k * 8, 8)], stage)
            pltpu.sync_copy(stage, S_o.at[cid, pl.ds(s0 + k * 8, 8)])
            return carry
        lax.fori_loop(0, rp // 8, outS, 0)
        if with_d:
            pltpu.sync_copy(dsh.at[pl.ds(s0, rp)], dlin.at[pl.ds(0, rp)])
            pltpu.sync_copy(dlin.at[pl.ds(0, rp)],
                            d_o.at[pl.ds(cid * NPAD + s0, rp)])

    return spmm


_SPMM_CACHE = {}


def _spmm(width, with_d):
    key = (width, with_d)
    if key not in _SPMM_CACHE:
        _SPMM_CACHE[key] = _make_spmm(width, with_d)
    return _SPMM_CACHE[key]


def kernel(feat, edges, adj, W1, b1, W2, b2, Wr, br, G1W, G1b, G2W, G2b,
           GfW, Gfb, C1W, C1b, C2W, C2b):
    row = lambda v: v.reshape(1, -1)
    y0 = _mm(feat, W1)
    x1 = _adjmm_relu(adj, y0, row(b1))
    z1 = _mm(x1, W2)
    x = _adjmm_relu(adj, z1, row(b2))
    noise = jax.random.normal(jax.random.key(7), (N, LAT), jnp.float32)
    degree, gen_feat = _tail(x, noise, Wr, row(br), G1W, row(G1b),
                             G2W, row(G2b), GfW, row(Gfb))
    deg_k = jnp.clip(degree.reshape(-1), 0, NUM_PRED).astype(jnp.int32)
    dkf = deg_k.astype(jnp.float32).reshape(N, 1)
    src, dst = _edge_slots(edges)
    y1top = _mm(feat, C1W)
    y1gen = _mm(gen_feat.reshape(-1, FEAT), C1W).reshape(N, NUM_PRED * HID)
    S1p, d1p = _spmm(HID, True)(src, dst, y1top)
    S1 = (S1p[0] + S1p[1])[:N]
    d1p = d1p.reshape(2, NPAD)
    d = ((d1p[0] + d1p[1])[:N] + 1.0
         + deg_k.astype(jnp.float32)).reshape(N, 1)
    h1top, h1gen = _assemble(S1, y1top, y1gen, dkf, d, row(C1b), HID)
    h1 = jnp.concatenate([h1top, h1gen.reshape(-1, HID)], axis=0)
    y2 = _mm(h1, C2W)
    y2top, y2gen = y2[:N], y2[N:].reshape(N, NUM_PRED * NCLS)
    S2p, = _spmm(NCLS, False)(src, dst, y2top)
    S2 = (S2p[0] + S2p[1])[:N]
    h2top, h2gen = _assemble(S2, y2top, y2gen, dkf, d, row(C2b), NCLS)
    h = jnp.concatenate([h2top, h2gen.reshape(-1, NCLS)], axis=0)
    return (degree, gen_feat, h)


# speedup vs baseline: 1.2382x; 1.1554x over previous
import functools

import jax
import jax.numpy as jnp
from jax import lax
from jax.experimental import pallas as pl
from jax.experimental.pallas import tpu as pltpu
from jax.experimental.pallas import tpu_sc as plsc

N = 10000
FEAT = 128
HID = 64
LAT = 64
NUM_PRED = 5
NCLS = 16
E = 320000
BM = 400


def _mm_body(a_ref, b_ref, o_ref):
    o_ref[...] = jnp.dot(a_ref[...], b_ref[...],
                         preferred_element_type=jnp.float32)


def _mm(a, b):
    n, k = a.shape
    _, m = b.shape
    return pl.pallas_call(
        _mm_body,
        grid=(n // BM,),
        in_specs=[pl.BlockSpec((BM, k), lambda i: (i, 0)),
                  pl.BlockSpec((k, m), lambda i: (0, 0))],
        out_specs=pl.BlockSpec((BM, m), lambda i: (i, 0)),
        out_shape=jax.ShapeDtypeStruct((n, m), jnp.float32),
    )(a, b)


def _adjmm_body(adj_ref, y_ref, b_ref, o_ref):
    acc = jnp.dot(adj_ref[...], y_ref[...],
                  preferred_element_type=jnp.float32)
    o_ref[...] = jnp.maximum(acc + b_ref[...], 0.0)


def _adjmm_relu(adj, y, bias):
    n = adj.shape[0]
    m = y.shape[1]
    return pl.pallas_call(
        _adjmm_body,
        grid=(n // BM,),
        in_specs=[pl.BlockSpec((BM, n), lambda i: (i, 0)),
                  pl.BlockSpec((n, m), lambda i: (0, 0)),
                  pl.BlockSpec((1, m), lambda i: (0, 0))],
        out_specs=pl.BlockSpec((BM, m), lambda i: (i, 0)),
        out_shape=jax.ShapeDtypeStruct((n, m), jnp.float32),
    )(adj, y, bias)


def _tail_body(x_ref, nz_ref, wr_ref, br_ref, g1w_ref, g1b_ref, g2w_ref,
               g2b_ref, gfw_ref, gfb_ref, deg_ref, gen_ref):
    x = x_ref[...]
    deg_ref[...] = jnp.maximum(
        jnp.dot(x, wr_ref[...], preferred_element_type=jnp.float32)
        + br_ref[...], 0.0)
    g = x + nz_ref[...]
    g = jnp.maximum(
        jnp.dot(g, g1w_ref[...], preferred_element_type=jnp.float32)
        + g1b_ref[...], 0.0)
    g = jnp.maximum(
        jnp.dot(g, g2w_ref[...], preferred_element_type=jnp.float32)
        + g2b_ref[...], 0.0)
    gen_ref[...] = jnp.tanh(
        jnp.dot(g, gfw_ref[...], preferred_element_type=jnp.float32)
        + gfb_ref[...])


def _tail(x, noise, Wr, br, G1W, G1b, G2W, G2b, GfW, Gfb):
    outs = (jax.ShapeDtypeStruct((N, 1), jnp.float32),
            jax.ShapeDtypeStruct((N, NUM_PRED * FEAT), jnp.float32))
    return pl.pallas_call(
        _tail_body,
        grid=(N // BM,),
        in_specs=[pl.BlockSpec((BM, LAT), lambda i: (i, 0)),
                  pl.BlockSpec((BM, LAT), lambda i: (i, 0)),
                  pl.BlockSpec((LAT, 1), lambda i: (0, 0)),
                  pl.BlockSpec((1, 1), lambda i: (0, 0)),
                  pl.BlockSpec((LAT, 256), lambda i: (0, 0)),
                  pl.BlockSpec((1, 256), lambda i: (0, 0)),
                  pl.BlockSpec((256, 2048), lambda i: (0, 0)),
                  pl.BlockSpec((1, 2048), lambda i: (0, 0)),
                  pl.BlockSpec((2048, NUM_PRED * FEAT), lambda i: (0, 0)),
                  pl.BlockSpec((1, NUM_PRED * FEAT), lambda i: (0, 0))],
        out_specs=(pl.BlockSpec((BM, 1), lambda i: (i, 0)),
                   pl.BlockSpec((BM, NUM_PRED * FEAT), lambda i: (i, 0))),
        out_shape=outs,
    )(x, noise, Wr, br, G1W, G1b, G2W, G2b, GfW, Gfb)


def _asm_body(s_ref, yt_ref, yg_ref, dk_ref, d_ref, b_ref, ht_ref, hg_ref,
              *, width):
    dk = dk_ref[...]
    yt = yt_ref[...]
    s = s_ref[...] + yt
    for j in range(NUM_PRED):
        ygj = yg_ref[:, j * width:(j + 1) * width]
        v = (dk > float(j)).astype(jnp.float32)
        s = s + v * ygj
        hg_ref[:, j * width:(j + 1) * width] = jnp.maximum(
            (ygj + v * yt) / (1.0 + v) + b_ref[...], 0.0)
    ht_ref[...] = jnp.maximum(s / d_ref[...] + b_ref[...], 0.0)


def _assemble(S, ytop, ygen, dkf, d, bias, width):
    outs = (jax.ShapeDtypeStruct((N, width), jnp.float32),
            jax.ShapeDtypeStruct((N, NUM_PRED * width), jnp.float32))
    return pl.pallas_call(
        functools.partial(_asm_body, width=width),
        grid=(N // BM,),
        in_specs=[pl.BlockSpec((BM, width), lambda i: (i, 0)),
                  pl.BlockSpec((BM, width), lambda i: (i, 0)),
                  pl.BlockSpec((BM, NUM_PRED * width), lambda i: (i, 0)),
                  pl.BlockSpec((BM, 1), lambda i: (i, 0)),
                  pl.BlockSpec((BM, 1), lambda i: (i, 0)),
                  pl.BlockSpec((1, width), lambda i: (0, 0))],
        out_specs=(pl.BlockSpec((BM, width), lambda i: (i, 0)),
                   pl.BlockSpec((BM, NUM_PRED * width), lambda i: (i, 0))),
        out_shape=outs,
    )(S, ytop, ygen, dkf, d, bias)


NDUM = 64
NPAD = 10112
EP = 655360
ROWS128 = EP // 128
TILES = 32
TROWS = ROWS128 // TILES
CHUNKS = TROWS // 4


def _edge_slots(edges):
    r = edges[:, 0].astype(jnp.int32)
    c = edges[:, 1].astype(jnp.int32)
    kmin = jnp.minimum(r, c)
    kmax = jnp.maximum(r, c)
    dirb = (r > c).astype(jnp.int32)
    key2 = (kmin * N + kmax) * 2 + dirb
    (sk,) = jax.lax.sort((key2,), dimension=0, is_stable=False, num_keys=1)
    pair = sk >> 1
    dir1 = (sk & 1) == 1
    pos = jnp.arange(E, dtype=jnp.int32)
    F1 = jnp.concatenate([jnp.array([True]), pair[1:] != pair[:-1]])
    F2 = jnp.concatenate([jnp.array([True]), sk[1:] != sk[:-1]])
    BIG = jnp.int32(E)
    s1 = jax.lax.cummax(jnp.where(F1, pos, -1))
    a1 = jnp.where(F1, pos, BIG)
    suf1 = jnp.flip(jax.lax.cummin(jnp.flip(a1)))
    e1 = jnp.concatenate([suf1[1:], jnp.array([E], jnp.int32)])
    am = jnp.where(F2 & dir1, pos, BIG)
    msuf = jnp.flip(jax.lax.cummin(jnp.flip(am)))
    m = jnp.minimum(msuf, e1)
    maxc_head = jnp.maximum(m - pos, e1 - m)
    maxc = jnp.take(maxc_head, s1)
    act1 = (pos - s1) < maxc
    kminS = pair // N
    kmaxS = pair % N
    act2 = act1 & (kminS != kmaxS)
    dummy = N + (pos % NDUM)
    srcA = kmaxS
    dstA = jnp.where(act1, kminS, dummy)
    srcB = kminS
    dstB = jnp.where(act2, kmaxS, dummy)
    npad = EP - 2 * E
    padi = jnp.arange(npad, dtype=jnp.int32)
    src = jnp.concatenate([srcA, srcB, jnp.zeros((npad,), jnp.int32)])
    dst = jnp.concatenate([dstA, dstB, N + padi % NDUM])
    return src.reshape(ROWS128, 128), dst.reshape(ROWS128, 128)


def _make_spmm(width, with_d):
    mesh = plsc.VectorSubcoreMesh(core_axis_name="c", subcore_axis_name="s")
    out_type = [jax.ShapeDtypeStruct((2, NPAD, width), jnp.float32)]
    if with_d:
        out_type.append(jax.ShapeDtypeStruct((2 * NPAD,), jnp.float32))
    scratch = [pltpu.VMEM((2, 4, 128), jnp.int32),
               pltpu.VMEM((2, 4, 128), jnp.int32),
               pltpu.VMEM((2, 4, 128, width), jnp.float32),
               pltpu.VMEM((128,), jnp.float32),
               pltpu.VMEM((8, width), jnp.float32),
               pltpu.VMEM((640,), jnp.float32),
               pltpu.VMEM_SHARED((NPAD, width), jnp.float32),
               pltpu.VMEM_SHARED((NPAD,), jnp.float32),
               pltpu.SemaphoreType.DMA]
    vw = width // 16

    @functools.partial(
        pl.kernel, mesh=mesh, out_type=out_type, scratch_types=scratch,
        compiler_params=pltpu.CompilerParams(use_tc_tiling_on_sc=False))
    def spmm(src_h, dst_h, y_h, S_o, *rest):
        if with_d:
            d_o, srcv, dstv, rows, onesv, stage, dlin, Ssh, dsh, gsem = rest
        else:
            srcv, dstv, rows, onesv, stage, dlin, Ssh, dsh, gsem = rest
        cid = lax.axis_index("c")
        sid = lax.axis_index("s")
        rp = NPAD // 16
        s0 = sid * rp
        z16 = jnp.zeros((16,), jnp.float32)

        def zstage(i, carry):
            stage[i // vw, pl.ds((i % vw) * 16, 16)] = z16
            return carry
        lax.fori_loop(0, 8 * vw, zstage, 0)

        def zdlin(i, carry):
            dlin[pl.ds(i * 16, 16)] = z16
            return carry
        lax.fori_loop(0, 40, zdlin, 0)

        def zones(i, carry):
            onesv[pl.ds(i * 16, 16)] = jnp.ones((16,), jnp.float32)
            return carry
        lax.fori_loop(0, 8, zones, 0)

        def zS(k, carry):
            pltpu.sync_copy(stage, Ssh.at[pl.ds(s0 + k * 8, 8)])
            return carry
        lax.fori_loop(0, rp // 8, zS, 0)
        if with_d:
            pltpu.sync_copy(dlin.at[pl.ds(0, rp)], dsh.at[pl.ds(s0, rp)])
        plsc.subcore_barrier()
        base = (cid * 16 + sid) * TROWS

        def load(k, buf):
            r0 = base + k * 4
            pltpu.sync_copy(src_h.at[pl.ds(r0, 4)], srcv.at[buf])
            pltpu.sync_copy(dst_h.at[pl.ds(r0, 4)], dstv.at[buf])
            for j in range(4):
                pltpu.async_copy(y_h.at[srcv.at[buf, j]],
                                 rows.at[buf, j], gsem)

        load(0, 0)

        def chunk(k, carry):
            pb = lax.rem(k, 2)
            nb = lax.rem(k + 1, 2)

            @pl.when(k + 1 < CHUNKS)
            def _():
                load(k + 1, nb)

            for j in range(4):
                pltpu.make_async_copy(y_h.at[srcv.at[pb, j]],
                                      rows.at[pb, j], gsem).wait()
            for j in range(4):
                pltpu.sync_copy(rows.at[pb, j], Ssh.at[dstv.at[pb, j]],
                                add=True)
                if with_d:
                    pltpu.sync_copy(onesv, dsh.at[dstv.at[pb, j]], add=True)
            return carry

        lax.fori_loop(0, CHUNKS, chunk, 0)
        plsc.subcore_barrier()

        def outS(k, carry):
            pltpu.sync_copy(Ssh.at[pl.ds(s0 + k * 8, 8)], stage)
            pltpu.sync_copy(stage, S_o.at[cid, pl.ds(s0 + k * 8, 8)])
            return carry
        lax.fori_loop(0, rp // 8, outS, 0)
        if with_d:
            pltpu.sync_copy(dsh.at[pl.ds(s0, rp)], dlin.at[pl.ds(0, rp)])
            pltpu.sync_copy(dlin.at[pl.ds(0, rp)],
                            d_o.at[pl.ds(cid * NPAD + s0, rp)])

    return spmm


_SPMM_CACHE = {}


def _spmm(width, with_d):
    key = (width, with_d)
    if key not in _SPMM_CACHE:
        _SPMM_CACHE[key] = _make_spmm(width, with_d)
    return _SPMM_CACHE[key]


import numpy as _np
_NOISE = _np.asarray(
    jax.random.normal(jax.random.key(7), (N, LAT), jnp.float32))


def _noise():
    return jnp.asarray(_NOISE)


def kernel(feat, edges, adj, W1, b1, W2, b2, Wr, br, G1W, G1b, G2W, G2b,
           GfW, Gfb, C1W, C1b, C2W, C2b):
    row = lambda v: v.reshape(1, -1)
    y0 = _mm(feat, W1)
    x1 = _adjmm_relu(adj, y0, row(b1))
    z1 = _mm(x1, W2)
    x = _adjmm_relu(adj, z1, row(b2))
    degree, gen_feat = _tail(x, _noise(), Wr, row(br), G1W, row(G1b),
                             G2W, row(G2b), GfW, row(Gfb))
    deg_k = jnp.clip(degree.reshape(-1), 0, NUM_PRED).astype(jnp.int32)
    dkf = deg_k.astype(jnp.float32).reshape(N, 1)
    src, dst = _edge_slots(edges)
    y1top = _mm(feat, C1W)
    y1gen = _mm(gen_feat.reshape(-1, FEAT), C1W).reshape(N, NUM_PRED * HID)
    S1p, d1p = _spmm(HID, True)(src, dst, y1top)
    S1 = (S1p[0] + S1p[1])[:N]
    d1p = d1p.reshape(2, NPAD)
    d = ((d1p[0] + d1p[1])[:N] + 1.0
         + deg_k.astype(jnp.float32)).reshape(N, 1)
    h1top, h1gen = _assemble(S1, y1top, y1gen, dkf, d, row(C1b), HID)
    h1 = jnp.concatenate([h1top, h1gen.reshape(-1, HID)], axis=0)
    y2 = _mm(h1, C2W)
    y2top, y2gen = y2[:N], y2[N:].reshape(N, NUM_PRED * NCLS)
    S2p, = _spmm(NCLS, False)(src, dst, y2top)
    S2 = (S2p[0] + S2p[1])[:N]
    h2top, h2gen = _assemble(S2, y2top, y2gen, dkf, d, row(C2b), NCLS)
    h = jnp.concatenate([h2top, h2gen.reshape(-1, NCLS)], axis=0)
    return (degree, gen_feat, h)


# speedup vs baseline: 1.2545x; 1.0131x over previous
import functools

import jax
import jax.numpy as jnp
from jax import lax
from jax.experimental import pallas as pl
from jax.experimental.pallas import tpu as pltpu
from jax.experimental.pallas import tpu_sc as plsc

N = 10000
FEAT = 128
HID = 64
LAT = 64
NUM_PRED = 5
NCLS = 16
E = 320000
BM = 400


def _mm_body(a_ref, b_ref, o_ref):
    o_ref[...] = jnp.dot(a_ref[...], b_ref[...],
                         preferred_element_type=jnp.float32)


def _mm(a, b):
    n, k = a.shape
    _, m = b.shape
    return pl.pallas_call(
        _mm_body,
        grid=(n // BM,),
        in_specs=[pl.BlockSpec((BM, k), lambda i: (i, 0)),
                  pl.BlockSpec((k, m), lambda i: (0, 0))],
        out_specs=pl.BlockSpec((BM, m), lambda i: (i, 0)),
        out_shape=jax.ShapeDtypeStruct((n, m), jnp.float32),
    )(a, b)


def _mm2_body(a_ref, b1_ref, b2_ref, o1_ref, o2_ref):
    a = a_ref[...]
    o1_ref[...] = jnp.dot(a, b1_ref[...], preferred_element_type=jnp.float32)
    o2_ref[...] = jnp.dot(a, b2_ref[...], preferred_element_type=jnp.float32)


def _mm2(a, b1, b2):
    n, k = a.shape
    m1, m2 = b1.shape[1], b2.shape[1]
    return pl.pallas_call(
        _mm2_body,
        grid=(n // BM,),
        in_specs=[pl.BlockSpec((BM, k), lambda i: (i, 0)),
                  pl.BlockSpec((k, m1), lambda i: (0, 0)),
                  pl.BlockSpec((k, m2), lambda i: (0, 0))],
        out_specs=(pl.BlockSpec((BM, m1), lambda i: (i, 0)),
                   pl.BlockSpec((BM, m2), lambda i: (i, 0))),
        out_shape=(jax.ShapeDtypeStruct((n, m1), jnp.float32),
                   jax.ShapeDtypeStruct((n, m2), jnp.float32)),
    )(a, b1, b2)


def _adjmm_body(adj_ref, y_ref, b_ref, o_ref):
    acc = jnp.dot(adj_ref[...], y_ref[...],
                  preferred_element_type=jnp.float32)
    o_ref[...] = jnp.maximum(acc + b_ref[...], 0.0)


def _adjmm_relu(adj, y, bias):
    n = adj.shape[0]
    m = y.shape[1]
    return pl.pallas_call(
        _adjmm_body,
        grid=(n // BM,),
        in_specs=[pl.BlockSpec((BM, n), lambda i: (i, 0)),
                  pl.BlockSpec((n, m), lambda i: (0, 0)),
                  pl.BlockSpec((1, m), lambda i: (0, 0))],
        out_specs=pl.BlockSpec((BM, m), lambda i: (i, 0)),
        out_shape=jax.ShapeDtypeStruct((n, m), jnp.float32),
    )(adj, y, bias)


def _adjmm_relu_mm_body(adj_ref, y_ref, b_ref, w_ref, o_ref):
    acc = jnp.dot(adj_ref[...], y_ref[...],
                  preferred_element_type=jnp.float32)
    x = jnp.maximum(acc + b_ref[...], 0.0)
    o_ref[...] = jnp.dot(x, w_ref[...], preferred_element_type=jnp.float32)


def _adjmm_relu_mm(adj, y, bias, w):
    n = adj.shape[0]
    m = y.shape[1]
    m2 = w.shape[1]
    return pl.pallas_call(
        _adjmm_relu_mm_body,
        grid=(n // BM,),
        in_specs=[pl.BlockSpec((BM, n), lambda i: (i, 0)),
                  pl.BlockSpec((n, m), lambda i: (0, 0)),
                  pl.BlockSpec((1, m), lambda i: (0, 0)),
                  pl.BlockSpec((m, m2), lambda i: (0, 0))],
        out_specs=pl.BlockSpec((BM, m2), lambda i: (i, 0)),
        out_shape=jax.ShapeDtypeStruct((n, m2), jnp.float32),
    )(adj, y, bias, w)


def _tail_body(x_ref, nz_ref, wr_ref, br_ref, g1w_ref, g1b_ref, g2w_ref,
               g2b_ref, gfw_ref, gfb_ref, deg_ref, gen_ref):
    x = x_ref[...]
    deg_ref[...] = jnp.maximum(
        jnp.dot(x, wr_ref[...], preferred_element_type=jnp.float32)
        + br_ref[...], 0.0)
    g = x + nz_ref[...]
    g = jnp.maximum(
        jnp.dot(g, g1w_ref[...], preferred_element_type=jnp.float32)
        + g1b_ref[...], 0.0)
    g = jnp.maximum(
        jnp.dot(g, g2w_ref[...], preferred_element_type=jnp.float32)
        + g2b_ref[...], 0.0)
    gen_ref[...] = jnp.tanh(
        jnp.dot(g, gfw_ref[...], preferred_element_type=jnp.float32)
        + gfb_ref[...])


def _tail(x, noise, Wr, br, G1W, G1b, G2W, G2b, GfW, Gfb):
    outs = (jax.ShapeDtypeStruct((N, 1), jnp.float32),
            jax.ShapeDtypeStruct((N, NUM_PRED * FEAT), jnp.float32))
    return pl.pallas_call(
        _tail_body,
        grid=(N // BM,),
        in_specs=[pl.BlockSpec((BM, LAT), lambda i: (i, 0)),
                  pl.BlockSpec((BM, LAT), lambda i: (i, 0)),
                  pl.BlockSpec((LAT, 1), lambda i: (0, 0)),
                  pl.BlockSpec((1, 1), lambda i: (0, 0)),
                  pl.BlockSpec((LAT, 256), lambda i: (0, 0)),
                  pl.BlockSpec((1, 256), lambda i: (0, 0)),
                  pl.BlockSpec((256, 2048), lambda i: (0, 0)),
                  pl.BlockSpec((1, 2048), lambda i: (0, 0)),
                  pl.BlockSpec((2048, NUM_PRED * FEAT), lambda i: (0, 0)),
                  pl.BlockSpec((1, NUM_PRED * FEAT), lambda i: (0, 0))],
        out_specs=(pl.BlockSpec((BM, 1), lambda i: (i, 0)),
                   pl.BlockSpec((BM, NUM_PRED * FEAT), lambda i: (i, 0))),
        out_shape=outs,
    )(x, noise, Wr, br, G1W, G1b, G2W, G2b, GfW, Gfb)


def _asm_body(s_ref, yt_ref, yg_ref, dk_ref, d_ref, b_ref, ht_ref, hg_ref,
              *, width):
    dk = dk_ref[...]
    yt = yt_ref[...]
    s = s_ref[...] + yt
    for j in range(NUM_PRED):
        ygj = yg_ref[:, j * width:(j + 1) * width]
        v = (dk > float(j)).astype(jnp.float32)
        s = s + v * ygj
        hg_ref[:, j * width:(j + 1) * width] = jnp.maximum(
            (ygj + v * yt) / (1.0 + v) + b_ref[...], 0.0)
    ht_ref[...] = jnp.maximum(s / d_ref[...] + b_ref[...], 0.0)


def _assemble(S, ytop, ygen, dkf, d, bias, width):
    outs = (jax.ShapeDtypeStruct((N, width), jnp.float32),
            jax.ShapeDtypeStruct((N, NUM_PRED * width), jnp.float32))
    return pl.pallas_call(
        functools.partial(_asm_body, width=width),
        grid=(N // BM,),
        in_specs=[pl.BlockSpec((BM, width), lambda i: (i, 0)),
                  pl.BlockSpec((BM, width), lambda i: (i, 0)),
                  pl.BlockSpec((BM, NUM_PRED * width), lambda i: (i, 0)),
                  pl.BlockSpec((BM, 1), lambda i: (i, 0)),
                  pl.BlockSpec((BM, 1), lambda i: (i, 0)),
                  pl.BlockSpec((1, width), lambda i: (0, 0))],
        out_specs=(pl.BlockSpec((BM, width), lambda i: (i, 0)),
                   pl.BlockSpec((BM, NUM_PRED * width), lambda i: (i, 0))),
        out_shape=outs,
    )(S, ytop, ygen, dkf, d, bias)


NDUM = 64
NPAD = 10112
EP = 655360
ROWS128 = EP // 128
TILES = 32
TROWS = ROWS128 // TILES
CHUNKS = TROWS // 4


def _edge_slots(edges):
    r = edges[:, 0].astype(jnp.int32)
    c = edges[:, 1].astype(jnp.int32)
    kmin = jnp.minimum(r, c)
    kmax = jnp.maximum(r, c)
    dirb = (r > c).astype(jnp.int32)
    key2 = (kmin * N + kmax) * 2 + dirb
    (sk,) = jax.lax.sort((key2,), dimension=0, is_stable=False, num_keys=1)
    pair = sk >> 1
    dir1 = (sk & 1) == 1
    pos = jnp.arange(E, dtype=jnp.int32)
    F1 = jnp.concatenate([jnp.array([True]), pair[1:] != pair[:-1]])
    F2 = jnp.concatenate([jnp.array([True]), sk[1:] != sk[:-1]])
    BIG = jnp.int32(E)
    s1 = jax.lax.cummax(jnp.where(F1, pos, -1))
    a1 = jnp.where(F1, pos, BIG)
    suf1 = jnp.flip(jax.lax.cummin(jnp.flip(a1)))
    e1 = jnp.concatenate([suf1[1:], jnp.array([E], jnp.int32)])
    am = jnp.where(F2 & dir1, pos, BIG)
    msuf = jnp.flip(jax.lax.cummin(jnp.flip(am)))
    m = jnp.minimum(msuf, e1)
    maxc_head = jnp.maximum(m - pos, e1 - m)
    maxc = jnp.take(maxc_head, s1)
    act1 = (pos - s1) < maxc
    kminS = pair // N
    kmaxS = pair % N
    act2 = act1 & (kminS != kmaxS)
    dummy = N + (pos % NDUM)
    srcA = kmaxS
    dstA = jnp.where(act1, kminS, dummy)
    srcB = kminS
    dstB = jnp.where(act2, kmaxS, dummy)
    npad = EP - 2 * E
    padi = jnp.arange(npad, dtype=jnp.int32)
    src = jnp.concatenate([srcA, srcB, jnp.zeros((npad,), jnp.int32)])
    dst = jnp.concatenate([dstA, dstB, N + padi % NDUM])
    return src.reshape(ROWS128, 128), dst.reshape(ROWS128, 128)


def _make_spmm(width, with_d):
    mesh = plsc.VectorSubcoreMesh(core_axis_name="c", subcore_axis_name="s")
    out_type = [jax.ShapeDtypeStruct((2, NPAD, width), jnp.float32)]
    if with_d:
        out_type.append(jax.ShapeDtypeStruct((2 * NPAD,), jnp.float32))
    scratch = [pltpu.VMEM((2, 4, 128), jnp.int32),
               pltpu.VMEM((2, 4, 128), jnp.int32),
               pltpu.VMEM((2, 4, 128, width), jnp.float32),
               pltpu.VMEM((128,), jnp.float32),
               pltpu.VMEM((8, width), jnp.float32),
               pltpu.VMEM((640,), jnp.float32),
               pltpu.VMEM_SHARED((NPAD, width), jnp.float32),
               pltpu.VMEM_SHARED((NPAD,), jnp.float32),
               pltpu.SemaphoreType.DMA]
    vw = width // 16

    @functools.partial(
        pl.kernel, mesh=mesh, out_type=out_type, scratch_types=scratch,
        compiler_params=pltpu.CompilerParams(use_tc_tiling_on_sc=False))
    def spmm(src_h, dst_h, y_h, S_o, *rest):
        if with_d:
            d_o, srcv, dstv, rows, onesv, stage, dlin, Ssh, dsh, gsem = rest
        else:
            srcv, dstv, rows, onesv, stage, dlin, Ssh, dsh, gsem = rest
        cid = lax.axis_index("c")
        sid = lax.axis_index("s")
        rp = NPAD // 16
        s0 = sid * rp
        z16 = jnp.zeros((16,), jnp.float32)

        def zstage(i, carry):
            stage[i // vw, pl.ds((i % vw) * 16, 16)] = z16
            return carry
        lax.fori_loop(0, 8 * vw, zstage, 0)

        def zdlin(i, carry):
            dlin[pl.ds(i * 16, 16)] = z16
            return carry
        lax.fori_loop(0, 40, zdlin, 0)

        def zones(i, carry):
            onesv[pl.ds(i * 16, 16)] = jnp.ones((16,), jnp.float32)
            return carry
        lax.fori_loop(0, 8, zones, 0)

        def zS(k, carry):
            pltpu.sync_copy(stage, Ssh.at[pl.ds(s0 + k * 8, 8)])
            return carry
        lax.fori_loop(0, rp // 8, zS, 0)
        if with_d:
            pltpu.sync_copy(dlin.at[pl.ds(0, rp)], dsh.at[pl.ds(s0, rp)])
        plsc.subcore_barrier()
        base = (cid * 16 + sid) * TROWS

        def load(k, buf):
            r0 = base + k * 4
            pltpu.sync_copy(src_h.at[pl.ds(r0, 4)], srcv.at[buf])
            pltpu.sync_copy(dst_h.at[pl.ds(r0, 4)], dstv.at[buf])
            for j in range(4):
                pltpu.async_copy(y_h.at[srcv.at[buf, j]],
                                 rows.at[buf, j], gsem)

        load(0, 0)

        def chunk(k, carry):
            pb = lax.rem(k, 2)
            nb = lax.rem(k + 1, 2)

            @pl.when(k + 1 < CHUNKS)
            def _():
                load(k + 1, nb)

            for j in range(4):
                pltpu.make_async_copy(y_h.at[srcv.at[pb, j]],
                                      rows.at[pb, j], gsem).wait()
            for j in range(4):
                pltpu.sync_copy(rows.at[pb, j], Ssh.at[dstv.at[pb, j]],
                                add=True)
                if with_d:
                    pltpu.sync_copy(onesv, dsh.at[dstv.at[pb, j]], add=True)
            return carry

        lax.fori_loop(0, CHUNKS, chunk, 0)
        plsc.subcore_barrier()

        def outS(k, carry):
            pltpu.sync_copy(Ssh.at[pl.ds(s0 + k * 8, 8)], stage)
            pltpu.sync_copy(stage, S_o.at[cid, pl.ds(s0 + k * 8, 8)])
            return carry
        lax.fori_loop(0, rp // 8, outS, 0)
        if with_d:
            pltpu.sync_copy(dsh.at[pl.ds(s0, rp)], dlin.at[pl.ds(0, rp)])
            pltpu.sync_copy(dlin.at[pl.ds(0, rp)],
                            d_o.at[pl.ds(cid * NPAD + s0, rp)])

    return spmm


_SPMM_CACHE = {}


def _spmm(width, with_d):
    key = (width, with_d)
    if key not in _SPMM_CACHE:
        _SPMM_CACHE[key] = _make_spmm(width, with_d)
    return _SPMM_CACHE[key]


import numpy as _np
_NOISE = _np.asarray(
    jax.random.normal(jax.random.key(7), (N, LAT), jnp.float32))


def _noise():
    return jnp.asarray(_NOISE)


def kernel(feat, edges, adj, W1, b1, W2, b2, Wr, br, G1W, G1b, G2W, G2b,
           GfW, Gfb, C1W, C1b, C2W, C2b):
    row = lambda v: v.reshape(1, -1)
    y0, y1top = _mm2(feat, W1, C1W)
    z1 = _adjmm_relu_mm(adj, y0, row(b1), W2)
    x = _adjmm_relu(adj, z1, row(b2))
    degree, gen_feat = _tail(x, _noise(), Wr, row(br), G1W, row(G1b),
                             G2W, row(G2b), GfW, row(Gfb))
    deg_k = jnp.clip(degree.reshape(-1), 0, NUM_PRED).astype(jnp.int32)
    dkf = deg_k.astype(jnp.float32).reshape(N, 1)
    src, dst = _edge_slots(edges)
    y1gen = _mm(gen_feat.reshape(-1, FEAT), C1W).reshape(N, NUM_PRED * HID)
    S1p, d1p = _spmm(HID, True)(src, dst, y1top)
    S1 = (S1p[0] + S1p[1])[:N]
    d1p = d1p.reshape(2, NPAD)
    d = ((d1p[0] + d1p[1])[:N] + 1.0
         + deg_k.astype(jnp.float32)).reshape(N, 1)
    h1top, h1gen = _assemble(S1, y1top, y1gen, dkf, d, row(C1b), HID)
    h1 = jnp.concatenate([h1top, h1gen.reshape(-1, HID)], axis=0)
    y2 = _mm(h1, C2W)
    y2top, y2gen = y2[:N], y2[N:].reshape(N, NUM_PRED * NCLS)
    S2p, = _spmm(NCLS, False)(src, dst, y2top)
    S2 = (S2p[0] + S2p[1])[:N]
    h2top, h2gen = _assemble(S2, y2top, y2gen, dkf, d, row(C2b), NCLS)
    h = jnp.concatenate([h2top, h2gen.reshape(-1, NCLS)], axis=0)
    return (degree, gen_feat, h)
